# speedup vs baseline: 5.7676x; 5.7676x over previous
import functools

import jax
import jax.numpy as jnp
from jax import lax
from jax.experimental import pallas as pl
from jax.experimental.pallas import tpu as pltpu
from jax.experimental.pallas import tpu_sc as plsc

VOCAB = 100000
D = 128

NC = 2
NS = 16
NW = NC * NS

B_SEQ = 1024
L_SEQ = 200
N_TOK = B_SEQ * L_SEQ
N_PER_W = N_TOK // NW
CHUNK = 128
NCHUNK = N_PER_W // CHUNK


def _make_gather():
    mesh = plsc.VectorSubcoreMesh(core_axis_name="c", subcore_axis_name="s")

    @functools.partial(
        pl.kernel,
        out_type=jax.ShapeDtypeStruct((NW, NCHUNK, CHUNK, D), jnp.float32),
        mesh=mesh,
        scratch_types=[
            pltpu.VMEM((NCHUNK, CHUNK), jnp.int32),
            pltpu.VMEM((CHUNK, D), jnp.float32),
            pltpu.SemaphoreType.DMA,
        ],
    )
    def gather_kernel(idx_hbm, table_hbm, out_hbm, idx_v, rows_v, sem):
        wid = lax.axis_index("s") * NC + lax.axis_index("c")
        pltpu.sync_copy(idx_hbm.at[wid], idx_v)

        def chunk_body(j, carry):
            pltpu.async_copy(table_hbm.at[idx_v.at[j]], rows_v, sem).wait()
            pltpu.sync_copy(rows_v, out_hbm.at[wid, j])
            return carry

        lax.fori_loop(0, NCHUNK, chunk_body, 0)

    return gather_kernel


_gather = _make_gather()


@jax.jit
def kernel(token_ids, table):
    idx = token_ids.astype(jnp.int32).reshape(NW, NCHUNK, CHUNK)
    out = _gather(idx, table)
    return out.reshape(B_SEQ, L_SEQ, D)


# speedup vs baseline: 7.2811x; 1.2624x over previous
import functools

import jax
import jax.numpy as jnp
from jax import lax
from jax.experimental import pallas as pl
from jax.experimental.pallas import tpu as pltpu
from jax.experimental.pallas import tpu_sc as plsc

VOCAB = 100000
D = 128

NC = 2
NS = 16
NW = NC * NS

B_SEQ = 1024
L_SEQ = 200
N_TOK = B_SEQ * L_SEQ
N_PER_W = N_TOK // NW
CHUNK = 128
NCHUNK = N_PER_W // CHUNK


def _make_gather():
    mesh = plsc.VectorSubcoreMesh(core_axis_name="c", subcore_axis_name="s")

    @functools.partial(
        pl.kernel,
        out_type=jax.ShapeDtypeStruct((NW, NCHUNK, CHUNK, D), jnp.float32),
        mesh=mesh,
        scratch_types=[
            pltpu.VMEM((NCHUNK, CHUNK), jnp.int32),
            pltpu.VMEM((CHUNK, D), jnp.float32),
            pltpu.VMEM((CHUNK, D), jnp.float32),
            pltpu.SemaphoreType.DMA,
            pltpu.SemaphoreType.DMA,
            pltpu.SemaphoreType.DMA,
            pltpu.SemaphoreType.DMA,
        ],
    )
    def gather_kernel(idx_hbm, table_hbm, out_hbm, idx_v, rows0, rows1,
                      g0, g1, o0, o1):
        wid = lax.axis_index("s") * NC + lax.axis_index("c")
        pltpu.sync_copy(idx_hbm.at[wid], idx_v)

        def wait_gather(buf, sem):
            pltpu.make_async_copy(table_hbm.at[idx_v.at[0]], buf, sem).wait()

        def wait_out(buf, sem):
            pltpu.make_async_copy(buf, out_hbm.at[wid, 0], sem).wait()

        pltpu.async_copy(table_hbm.at[idx_v.at[0]], rows0, g0)
        pltpu.async_copy(table_hbm.at[idx_v.at[1]], rows1, g1)
        wait_gather(rows0, g0)
        pltpu.async_copy(rows0, out_hbm.at[wid, 0], o0)
        wait_gather(rows1, g1)
        pltpu.async_copy(rows1, out_hbm.at[wid, 1], o1)

        @pl.loop(2, NCHUNK, step=2)
        def body(j):
            wait_out(rows0, o0)
            pltpu.async_copy(table_hbm.at[idx_v.at[j]], rows0, g0)
            wait_out(rows1, o1)
            pltpu.async_copy(table_hbm.at[idx_v.at[j + 1]], rows1, g1)
            wait_gather(rows0, g0)
            pltpu.async_copy(rows0, out_hbm.at[wid, j], o0)
            wait_gather(rows1, g1)
            pltpu.async_copy(rows1, out_hbm.at[wid, j + 1], o1)

        wait_out(rows0, o0)
        wait_out(rows1, o1)

    return gather_kernel


_gather = _make_gather()


@jax.jit
def kernel(token_ids, table):
    idx = token_ids.astype(jnp.int32).reshape(NW, NCHUNK, CHUNK)
    out = _gather(idx, table)
    return out.reshape(B_SEQ, L_SEQ, D)


# speedup vs baseline: 7.7868x; 1.0695x over previous
import functools

import jax
import jax.numpy as jnp
from jax import lax
from jax.experimental import pallas as pl
from jax.experimental.pallas import tpu as pltpu
from jax.experimental.pallas import tpu_sc as plsc

VOCAB = 100000
D = 128

NC = 2
NS = 16
NW = NC * NS

B_SEQ = 1024
L_SEQ = 200
N_TOK = B_SEQ * L_SEQ
N_PER_W = N_TOK // NW
CHUNK = 128
NCHUNK = N_PER_W // CHUNK


def _make_gather():
    mesh = plsc.VectorSubcoreMesh(core_axis_name="c", subcore_axis_name="s")

    NBUF = 5
    assert NCHUNK % NBUF == 0

    @functools.partial(
        pl.kernel,
        out_type=jax.ShapeDtypeStruct((NW, NCHUNK, CHUNK, D), jnp.float32),
        mesh=mesh,
        scratch_types=[
            pltpu.VMEM((NCHUNK, CHUNK), jnp.int32),
            [pltpu.VMEM((CHUNK, D), jnp.float32)] * NBUF,
            [pltpu.SemaphoreType.DMA] * NBUF,
            [pltpu.SemaphoreType.DMA] * NBUF,
        ],
    )
    def gather_kernel(idx_hbm, table_hbm, out_hbm, idx_v, rows, gs, os):
        wid = lax.axis_index("s") * NC + lax.axis_index("c")
        pltpu.sync_copy(idx_hbm.at[wid], idx_v)

        def wait_gather(b):
            pltpu.make_async_copy(
                table_hbm.at[idx_v.at[0]], rows[b], gs[b]).wait()

        def wait_out(b):
            pltpu.make_async_copy(rows[b], out_hbm.at[wid, 0], os[b]).wait()

        for b in range(NBUF):
            pltpu.async_copy(table_hbm.at[idx_v.at[b]], rows[b], gs[b])

        @pl.loop(0, NCHUNK - NBUF, step=NBUF)
        def body(j):
            for b in range(NBUF):
                wait_gather(b)
                pltpu.async_copy(rows[b], out_hbm.at[wid, j + b], os[b])
            for b in range(NBUF):
                wait_out(b)
                pltpu.async_copy(
                    table_hbm.at[idx_v.at[j + b + NBUF]], rows[b], gs[b])

        for b in range(NBUF):
            wait_gather(b)
            pltpu.async_copy(
                rows[b], out_hbm.at[wid, NCHUNK - NBUF + b], os[b])
        for b in range(NBUF):
            wait_out(b)

    return gather_kernel


_gather = _make_gather()


@jax.jit
def kernel(token_ids, table):
    idx = token_ids.astype(jnp.int32).reshape(NW, NCHUNK, CHUNK)
    out = _gather(idx, table)
    return out.reshape(B_SEQ, L_SEQ, D)


# speedup vs baseline: 7.8711x; 1.0108x over previous
import functools

import jax
import jax.numpy as jnp
from jax import lax
from jax.experimental import pallas as pl
from jax.experimental.pallas import tpu as pltpu
from jax.experimental.pallas import tpu_sc as plsc

VOCAB = 100000
D = 128

NC = 2
NS = 16
NW = NC * NS

B_SEQ = 1024
L_SEQ = 200
N_TOK = B_SEQ * L_SEQ
N_PER_W = N_TOK // NW
CHUNK = 64
NCHUNK = N_PER_W // CHUNK


def _make_gather():
    mesh = plsc.VectorSubcoreMesh(core_axis_name="c", subcore_axis_name="s")

    NBUF = 10
    assert NCHUNK % NBUF == 0

    @functools.partial(
        pl.kernel,
        out_type=jax.ShapeDtypeStruct((NW, NCHUNK, CHUNK, D), jnp.float32),
        mesh=mesh,
        scratch_types=[
            pltpu.VMEM((NCHUNK, CHUNK), jnp.int32),
            [pltpu.VMEM((CHUNK, D), jnp.float32)] * NBUF,
            [pltpu.SemaphoreType.DMA] * NBUF,
            [pltpu.SemaphoreType.DMA] * NBUF,
        ],
    )
    def gather_kernel(idx_hbm, table_hbm, out_hbm, idx_v, rows, gs, os):
        wid = lax.axis_index("s") * NC + lax.axis_index("c")
        pltpu.sync_copy(idx_hbm.at[wid], idx_v)

        def wait_gather(b):
            pltpu.make_async_copy(
                table_hbm.at[idx_v.at[0]], rows[b], gs[b]).wait()

        def wait_out(b):
            pltpu.make_async_copy(rows[b], out_hbm.at[wid, 0], os[b]).wait()

        for b in range(NBUF):
            pltpu.async_copy(table_hbm.at[idx_v.at[b]], rows[b], gs[b])

        @pl.loop(0, NCHUNK - NBUF, step=NBUF)
        def body(j):
            for b in range(NBUF):
                wait_gather(b)
                pltpu.async_copy(rows[b], out_hbm.at[wid, j + b], os[b])
            for b in range(NBUF):
                wait_out(b)
                pltpu.async_copy(
                    table_hbm.at[idx_v.at[j + b + NBUF]], rows[b], gs[b])

        for b in range(NBUF):
            wait_gather(b)
            pltpu.async_copy(
                rows[b], out_hbm.at[wid, NCHUNK - NBUF + b], os[b])
        for b in range(NBUF):
            wait_out(b)

    return gather_kernel


_gather = _make_gather()


@jax.jit
def kernel(token_ids, table):
    idx = token_ids.astype(jnp.int32).reshape(NW, NCHUNK, CHUNK)
    out = _gather(idx, table)
    return out.reshape(B_SEQ, L_SEQ, D)
